# Initial kernel scaffold; baseline (speedup 1.0000x reference)
import jax
import jax.numpy as jnp
from jax.experimental import pallas as pl


def kernel(x, edge_index, edge_attr, batch, emb_W, emb_b, c0_eW1, c0_eb1, c0_eW2, c0_eb2, c0_nW, c0_nb, bn0_g, bn0_b, c1_eW1, c1_eb1, c1_eW2, c1_eb2, c1_nW, c1_nb, bn1_g, bn1_b, p_W1, p_b1, p_W2, p_b2):
    raise NotImplementedError("write your pallas kernel here")


# speedup vs baseline: 1.0393x; 1.0393x over previous
import functools

import jax
import jax.numpy as jnp
from jax import lax
from jax.experimental import pallas as pl
from jax.experimental.pallas import tpu as pltpu
from jax.experimental.pallas import tpu_sc as plsc

H = 32
CHUNK = 128
SPAN = 20
NC = 2
NS = 16
NW = NC * NS


def _embed_body(x_ref, w_ref, b_ref, o_ref):
    o_ref[...] = x_ref[...] @ w_ref[...] + b_ref[...]


def _edge_body(ea_ref, xj_ref, w1_ref, b1_ref, w2f_ref, b2r_ref, o_ref):
    ea = ea_ref[...]
    xj = xj_ref[...]
    eh = jnp.maximum(ea @ w1_ref[...] + b1_ref[...], 0.0)
    op = jnp.concatenate([eh[:, c:c + 1] * xj for c in range(H)], axis=1)
    o_ref[...] = op @ w2f_ref[...] + xj @ b2r_ref[...]


def _node_common(acc_ref, nw_ref, nb_ref, g_ref, b_ref, n_nodes, npad):
    af = acc_ref[...]
    a = af[:n_nodes] + af[npad:npad + n_nodes]
    h = jnp.maximum(a @ nw_ref[...] + nb_ref[...], 0.0)
    mean = jnp.mean(h, axis=0, keepdims=True)
    var = jnp.mean((h - mean) ** 2, axis=0, keepdims=True)
    hn = (h - mean) / jnp.sqrt(var + 1e-5) * g_ref[...] + b_ref[...]
    return jnp.maximum(hn, 0.0)


def _node_body(acc_ref, nw_ref, nb_ref, g_ref, b_ref, o_ref, *, n_nodes, npad):
    o_ref[...] = _node_common(acc_ref, nw_ref, nb_ref, g_ref, b_ref,
                              n_nodes, npad)


def _final_body(acc_ref, nw_ref, nb_ref, g_ref, b_ref, batch_ref,
                pw1_ref, pb1_ref, pw2_ref, pb2_ref, o_ref, *,
                n_nodes, npad, ng):
    h = _node_common(acc_ref, nw_ref, nb_ref, g_ref, b_ref, n_nodes, npad)
    bvec = batch_ref[...]
    gid = lax.broadcasted_iota(jnp.int32, (ng, n_nodes), 0)
    oh = (gid == bvec).astype(jnp.float32)
    sums = oh @ h
    cnt = jnp.sum(oh, axis=1, keepdims=True)
    pooled = sums / jnp.maximum(cnt, 1.0)
    z = jnp.maximum(pooled @ pw1_ref[...] + pb1_ref[...], 0.0)
    o_ref[...] = z @ pw2_ref[...] + pb2_ref[...]


def _gather_call(h, src_r):
    ch = src_r.shape[0] * SPAN
    cpw = ch // NW
    nspan = cpw // SPAN
    mesh = plsc.VectorSubcoreMesh(core_axis_name="c", subcore_axis_name="s")

    @functools.partial(
        pl.kernel, mesh=mesh,
        out_type=jax.ShapeDtypeStruct((ch * CHUNK, H), jnp.float32),
        compiler_params=pltpu.CompilerParams(use_tc_tiling_on_sc=False),
        scratch_types=[
            pltpu.VMEM((SPAN, CHUNK), jnp.int32),
            pltpu.VMEM((SPAN * CHUNK, H), jnp.float32),
            pltpu.SemaphoreType.DMA,
        ],
    )
    def k(h_hbm, src_hbm, out_hbm, idx_v, rows_v, sem):
        wid = lax.axis_index("s") * NC + lax.axis_index("c")
        for span in range(nspan):
            crow = wid * cpw + span * SPAN
            pltpu.sync_copy(src_hbm.at[wid * nspan + span], idx_v)
            cps = [pltpu.async_copy(h_hbm.at[idx_v.at[j]],
                                    rows_v.at[pl.ds(j * CHUNK, CHUNK)], sem)
                   for j in range(SPAN)]
            for cp in cps:
                cp.wait()
            pltpu.sync_copy(rows_v,
                            out_hbm.at[pl.ds(crow * CHUNK, SPAN * CHUNK)])

    return k(h, src_r)


def _scatter_call(msg, dst_r, zeros_pad, npad):
    ch = dst_r.shape[0] * SPAN
    cpw = ch // NW
    nspan = cpw // SPAN
    zr = npad // NS
    mesh = plsc.VectorSubcoreMesh(core_axis_name="c", subcore_axis_name="s")

    @functools.partial(
        pl.kernel, mesh=mesh,
        out_type=jax.ShapeDtypeStruct((2 * npad, H), jnp.float32),
        compiler_params=pltpu.CompilerParams(use_tc_tiling_on_sc=False),
        scratch_types=[
            pltpu.VMEM((SPAN, CHUNK), jnp.int32),
            pltpu.VMEM((SPAN * CHUNK, H), jnp.float32),
            pltpu.VMEM_SHARED((npad, H), jnp.float32),
            pltpu.SemaphoreType.DMA,
        ],
    )
    def k(msg_hbm, dst_hbm, zero_hbm, out_hbm, idx_v, rows_v, acc, sem):
        cid = lax.axis_index("c")
        sid = lax.axis_index("s")
        wid = sid * NC + cid
        pltpu.sync_copy(zero_hbm.at[pl.ds(sid * zr, zr)],
                        acc.at[pl.ds(sid * zr, zr)])
        plsc.subcore_barrier()
        for span in range(nspan):
            crow = wid * cpw + span * SPAN
            pltpu.sync_copy(dst_hbm.at[wid * nspan + span], idx_v)
            pltpu.sync_copy(msg_hbm.at[pl.ds(crow * CHUNK, SPAN * CHUNK)],
                            rows_v)
            cps = [pltpu.async_copy(rows_v.at[pl.ds(j * CHUNK, CHUNK)],
                                    acc.at[idx_v.at[j]], sem, add=True)
                   for j in range(SPAN)]
            for cp in cps:
                cp.wait()
        plsc.subcore_barrier()
        pltpu.sync_copy(acc.at[pl.ds(sid * zr, zr)],
                        out_hbm.at[pl.ds(cid * npad + sid * zr, zr)])

    return k(msg, dst_r, zeros_pad)


def kernel(x, edge_index, edge_attr, batch, emb_W, emb_b,
           c0_eW1, c0_eb1, c0_eW2, c0_eb2, c0_nW, c0_nb, bn0_g, bn0_b,
           c1_eW1, c1_eb1, c1_eW2, c1_eb2, c1_nW, c1_nb, bn1_g, bn1_b,
           p_W1, p_b1, p_W2, p_b2):
    n_nodes, node_dim = x.shape
    e_edges = edge_index.shape[1]
    edge_dim = edge_attr.shape[1]
    ng = 64
    out_dim = p_W2.shape[1]

    step = NW * SPAN
    ch = ((e_edges + CHUNK - 1) // CHUNK + step - 1) // step * step
    ep = ch * CHUNK
    npad = ((n_nodes + 1) + 255) // 256 * 256

    src = edge_index[0]
    dst = edge_index[1]
    pad_e = ep - e_edges
    src_r = jnp.concatenate(
        [src, jnp.zeros((pad_e,), jnp.int32)]).reshape(ch // SPAN, SPAN, CHUNK)
    dst_r = jnp.concatenate(
        [dst, jnp.full((pad_e,), n_nodes, jnp.int32)]).reshape(
            ch // SPAN, SPAN, CHUNK)
    ea_p = jnp.concatenate(
        [edge_attr, jnp.zeros((pad_e, edge_dim), jnp.float32)], axis=0)
    zeros_pad = jnp.zeros((npad, H), jnp.float32)
    batch2 = batch.reshape(1, n_nodes)

    h = pl.pallas_call(
        _embed_body,
        out_shape=jax.ShapeDtypeStruct((n_nodes, H), jnp.float32),
    )(x, emb_W, emb_b.reshape(1, H))

    layers = [
        (c0_eW1, c0_eb1, c0_eW2, c0_eb2, c0_nW, c0_nb, bn0_g, bn0_b),
        (c1_eW1, c1_eb1, c1_eW2, c1_eb2, c1_nW, c1_nb, bn1_g, bn1_b),
    ]

    be = 2048
    grid_e = ep // be
    edge_call = pl.pallas_call(
        _edge_body,
        grid=(grid_e,),
        in_specs=[
            pl.BlockSpec((be, edge_dim), lambda i: (i, 0)),
            pl.BlockSpec((be, H), lambda i: (i, 0)),
            pl.BlockSpec((edge_dim, H), lambda i: (0, 0)),
            pl.BlockSpec((1, H), lambda i: (0, 0)),
            pl.BlockSpec((H * H, H), lambda i: (0, 0)),
            pl.BlockSpec((H, H), lambda i: (0, 0)),
        ],
        out_specs=pl.BlockSpec((be, H), lambda i: (i, 0)),
        out_shape=jax.ShapeDtypeStruct((ep, H), jnp.float32),
    )

    for li, (ew1, eb1, ew2, eb2, nw, nb, g, b) in enumerate(layers):
        xj = _gather_call(h, src_r)
        msg = edge_call(ea_p, xj, ew1, eb1.reshape(1, H),
                        ew2.reshape(H * H, H), eb2.reshape(H, H))
        acc = _scatter_call(msg, dst_r, zeros_pad, npad)
        if li == 0:
            h = pl.pallas_call(
                functools.partial(_node_body, n_nodes=n_nodes, npad=npad),
                out_shape=jax.ShapeDtypeStruct((n_nodes, H), jnp.float32),
            )(acc, nw, nb.reshape(1, H), g.reshape(1, H), b.reshape(1, H))
        else:
            out = pl.pallas_call(
                functools.partial(_final_body, n_nodes=n_nodes, npad=npad,
                                  ng=ng),
                out_shape=jax.ShapeDtypeStruct((ng, out_dim), jnp.float32),
            )(acc, nw, nb.reshape(1, H), g.reshape(1, H), b.reshape(1, H),
              batch2, p_W1, p_b1.reshape(1, -1), p_W2, p_b2.reshape(1, -1))
    return out


# speedup vs baseline: 3.9326x; 3.7839x over previous
import functools

import jax
import jax.numpy as jnp
from jax import lax
from jax.experimental import pallas as pl
from jax.experimental.pallas import tpu as pltpu
from jax.experimental.pallas import tpu_sc as plsc

H = 32
CHUNK = 128
SPAN = 20
NC = 2
NS = 16
NW = NC * NS


def _embed_body(x_ref, w_ref, b_ref, o_ref):
    o_ref[...] = x_ref[...] @ w_ref[...] + b_ref[...]


def _edge_body(eaT_ref, xj_ref, w1T_ref, b1T_ref, w2fT_ref, b2rT_ref, o_ref):
    xjT = xj_ref[...].T
    ehT = jnp.maximum(w1T_ref[...] @ eaT_ref[...] + b1T_ref[...], 0.0)
    opT = jnp.concatenate([ehT[c:c + 1, :] * xjT for c in range(H)], axis=0)
    msgT = w2fT_ref[...] @ opT + b2rT_ref[...] @ xjT
    o_ref[...] = msgT.T


def _node_common(acc_ref, nw_ref, nb_ref, g_ref, b_ref, n_nodes, npad):
    af = acc_ref[...]
    a = af[:n_nodes] + af[npad:npad + n_nodes]
    h = jnp.maximum(a @ nw_ref[...] + nb_ref[...], 0.0)
    mean = jnp.mean(h, axis=0, keepdims=True)
    var = jnp.mean((h - mean) ** 2, axis=0, keepdims=True)
    hn = (h - mean) / jnp.sqrt(var + 1e-5) * g_ref[...] + b_ref[...]
    return jnp.maximum(hn, 0.0)


def _node_body(acc_ref, nw_ref, nb_ref, g_ref, b_ref, o_ref, *, n_nodes, npad):
    o_ref[...] = _node_common(acc_ref, nw_ref, nb_ref, g_ref, b_ref,
                              n_nodes, npad)


def _final_body(acc_ref, nw_ref, nb_ref, g_ref, b_ref, batch_ref,
                pw1_ref, pb1_ref, pw2_ref, pb2_ref, o_ref, *,
                n_nodes, npad, ng):
    h = _node_common(acc_ref, nw_ref, nb_ref, g_ref, b_ref, n_nodes, npad)
    bvec = batch_ref[...]
    gid = lax.broadcasted_iota(jnp.int32, (ng, n_nodes), 0)
    oh = (gid == bvec).astype(jnp.float32)
    sums = oh @ h
    cnt = jnp.sum(oh, axis=1, keepdims=True)
    pooled = sums / jnp.maximum(cnt, 1.0)
    z = jnp.maximum(pooled @ pw1_ref[...] + pb1_ref[...], 0.0)
    o_ref[...] = z @ pw2_ref[...] + pb2_ref[...]


def _gather_call(h, src_r):
    ch = src_r.shape[0] * SPAN
    cpw = ch // NW
    nspan = cpw // SPAN
    mesh = plsc.VectorSubcoreMesh(core_axis_name="c", subcore_axis_name="s")

    @functools.partial(
        pl.kernel, mesh=mesh,
        out_type=jax.ShapeDtypeStruct((ch * CHUNK, H), jnp.float32),
        compiler_params=pltpu.CompilerParams(use_tc_tiling_on_sc=False),
        scratch_types=[
            pltpu.VMEM((SPAN, CHUNK), jnp.int32),
            pltpu.VMEM((SPAN * CHUNK, H), jnp.float32),
            pltpu.SemaphoreType.DMA,
        ],
    )
    def k(h_hbm, src_hbm, out_hbm, idx_v, rows_v, sem):
        wid = lax.axis_index("s") * NC + lax.axis_index("c")
        for span in range(nspan):
            crow = wid * cpw + span * SPAN
            pltpu.sync_copy(src_hbm.at[wid * nspan + span], idx_v)
            cps = [pltpu.async_copy(h_hbm.at[idx_v.at[j]],
                                    rows_v.at[pl.ds(j * CHUNK, CHUNK)], sem)
                   for j in range(SPAN)]
            for cp in cps:
                cp.wait()
            pltpu.sync_copy(rows_v,
                            out_hbm.at[pl.ds(crow * CHUNK, SPAN * CHUNK)])

    return k(h, src_r)


def _scatter_call(msg, dst_r, zeros_pad, npad):
    ch = dst_r.shape[0] * SPAN
    cpw = ch // NW
    nspan = cpw // SPAN
    zr = npad // NS
    mesh = plsc.VectorSubcoreMesh(core_axis_name="c", subcore_axis_name="s")

    @functools.partial(
        pl.kernel, mesh=mesh,
        out_type=jax.ShapeDtypeStruct((2 * npad, H), jnp.float32),
        compiler_params=pltpu.CompilerParams(use_tc_tiling_on_sc=False),
        scratch_types=[
            pltpu.VMEM((SPAN, CHUNK), jnp.int32),
            pltpu.VMEM((SPAN * CHUNK, H), jnp.float32),
            pltpu.VMEM_SHARED((npad, H), jnp.float32),
            pltpu.SemaphoreType.DMA,
        ],
    )
    def k(msg_hbm, dst_hbm, zero_hbm, out_hbm, idx_v, rows_v, acc, sem):
        cid = lax.axis_index("c")
        sid = lax.axis_index("s")
        wid = sid * NC + cid
        pltpu.sync_copy(zero_hbm.at[pl.ds(sid * zr, zr)],
                        acc.at[pl.ds(sid * zr, zr)])
        plsc.subcore_barrier()
        for span in range(nspan):
            crow = wid * cpw + span * SPAN
            pltpu.sync_copy(dst_hbm.at[wid * nspan + span], idx_v)
            pltpu.sync_copy(msg_hbm.at[pl.ds(crow * CHUNK, SPAN * CHUNK)],
                            rows_v)
            cps = [pltpu.async_copy(rows_v.at[pl.ds(j * CHUNK, CHUNK)],
                                    acc.at[idx_v.at[j]], sem, add=True)
                   for j in range(SPAN)]
            for cp in cps:
                cp.wait()
        plsc.subcore_barrier()
        pltpu.sync_copy(acc.at[pl.ds(sid * zr, zr)],
                        out_hbm.at[pl.ds(cid * npad + sid * zr, zr)])

    return k(msg, dst_r, zeros_pad)


def kernel(x, edge_index, edge_attr, batch, emb_W, emb_b,
           c0_eW1, c0_eb1, c0_eW2, c0_eb2, c0_nW, c0_nb, bn0_g, bn0_b,
           c1_eW1, c1_eb1, c1_eW2, c1_eb2, c1_nW, c1_nb, bn1_g, bn1_b,
           p_W1, p_b1, p_W2, p_b2):
    n_nodes, node_dim = x.shape
    e_edges = edge_index.shape[1]
    edge_dim = edge_attr.shape[1]
    ng = 64
    out_dim = p_W2.shape[1]

    step = NW * SPAN
    ch = ((e_edges + CHUNK - 1) // CHUNK + step - 1) // step * step
    ep = ch * CHUNK
    npad = ((n_nodes + 1) + 255) // 256 * 256

    src = edge_index[0]
    dst = edge_index[1]
    pad_e = ep - e_edges
    src_r = jnp.concatenate(
        [src, jnp.zeros((pad_e,), jnp.int32)]).reshape(ch // SPAN, SPAN, CHUNK)
    dst_r = jnp.concatenate(
        [dst, jnp.full((pad_e,), n_nodes, jnp.int32)]).reshape(
            ch // SPAN, SPAN, CHUNK)
    ea_t = jnp.concatenate(
        [edge_attr, jnp.zeros((pad_e, edge_dim), jnp.float32)], axis=0).T
    zeros_pad = jnp.zeros((npad, H), jnp.float32)
    batch2 = batch.reshape(1, n_nodes)

    h = pl.pallas_call(
        _embed_body,
        out_shape=jax.ShapeDtypeStruct((n_nodes, H), jnp.float32),
    )(x, emb_W, emb_b.reshape(1, H))

    layers = [
        (c0_eW1, c0_eb1, c0_eW2, c0_eb2, c0_nW, c0_nb, bn0_g, bn0_b),
        (c1_eW1, c1_eb1, c1_eW2, c1_eb2, c1_nW, c1_nb, bn1_g, bn1_b),
    ]

    be = 2048
    grid_e = ep // be
    edge_call = pl.pallas_call(
        _edge_body,
        grid=(grid_e,),
        in_specs=[
            pl.BlockSpec((edge_dim, be), lambda i: (0, i)),
            pl.BlockSpec((be, H), lambda i: (i, 0)),
            pl.BlockSpec((H, edge_dim), lambda i: (0, 0)),
            pl.BlockSpec((H, 1), lambda i: (0, 0)),
            pl.BlockSpec((H, H * H), lambda i: (0, 0)),
            pl.BlockSpec((H, H), lambda i: (0, 0)),
        ],
        out_specs=pl.BlockSpec((be, H), lambda i: (i, 0)),
        out_shape=jax.ShapeDtypeStruct((ep, H), jnp.float32),
    )

    for li, (ew1, eb1, ew2, eb2, nw, nb, g, b) in enumerate(layers):
        xj = _gather_call(h, src_r)
        w2ft = ew2.reshape(H, H, H).transpose(2, 0, 1).reshape(H, H * H)
        msg = edge_call(ea_t, xj, ew1.T, eb1.reshape(H, 1),
                        w2ft, eb2.reshape(H, H).T)
        acc = _scatter_call(msg, dst_r, zeros_pad, npad)
        if li == 0:
            h = pl.pallas_call(
                functools.partial(_node_body, n_nodes=n_nodes, npad=npad),
                out_shape=jax.ShapeDtypeStruct((n_nodes, H), jnp.float32),
            )(acc, nw, nb.reshape(1, H), g.reshape(1, H), b.reshape(1, H))
        else:
            out = pl.pallas_call(
                functools.partial(_final_body, n_nodes=n_nodes, npad=npad,
                                  ng=ng),
                out_shape=jax.ShapeDtypeStruct((ng, out_dim), jnp.float32),
            )(acc, nw, nb.reshape(1, H), g.reshape(1, H), b.reshape(1, H),
              batch2, p_W1, p_b1.reshape(1, -1), p_W2, p_b2.reshape(1, -1))
    return out


# speedup vs baseline: 5.6733x; 1.4426x over previous
import functools

import jax
import jax.numpy as jnp
from jax import lax
from jax.experimental import pallas as pl
from jax.experimental.pallas import tpu as pltpu
from jax.experimental.pallas import tpu_sc as plsc

H = 32
CHUNK = 128
SPAN = 20
NC = 2
NS = 16
NW = NC * NS


def _embed_body(x_ref, w_ref, b_ref, o_ref):
    o_ref[...] = x_ref[...] @ w_ref[...] + b_ref[...]


def _edge_body(eaT_ref, xj_ref, w1T_ref, b1T_ref, w2fT_ref, b2rT_ref, o_ref):
    xjT = xj_ref[:, :H].T
    ehT = jnp.maximum(w1T_ref[...] @ eaT_ref[...] + b1T_ref[...], 0.0)
    opT = jnp.concatenate([ehT[c:c + 1, :] * xjT for c in range(H)], axis=0)
    msgT = w2fT_ref[...] @ opT + b2rT_ref[...] @ xjT
    o_ref[:, :H] = msgT.T


def _node_common(acc_ref, nw_ref, nb_ref, g_ref, b_ref, n_nodes, npad):
    af = acc_ref[...]
    a = af[:n_nodes] + af[npad:npad + n_nodes]
    h = jnp.maximum(a @ nw_ref[...] + nb_ref[...], 0.0)
    mean = jnp.mean(h, axis=0, keepdims=True)
    var = jnp.mean((h - mean) ** 2, axis=0, keepdims=True)
    hn = (h - mean) / jnp.sqrt(var + 1e-5) * g_ref[...] + b_ref[...]
    return jnp.maximum(hn, 0.0)


def _node_body(acc_ref, nw_ref, nb_ref, g_ref, b_ref, o_ref, *, n_nodes, npad):
    o_ref[...] = _node_common(acc_ref, nw_ref, nb_ref, g_ref, b_ref,
                              n_nodes, npad)


def _final_body(acc_ref, nw_ref, nb_ref, g_ref, b_ref, batch_ref,
                pw1_ref, pb1_ref, pw2_ref, pb2_ref, o_ref, *,
                n_nodes, npad, ng):
    h = _node_common(acc_ref, nw_ref, nb_ref, g_ref, b_ref, n_nodes, npad)
    bvec = batch_ref[...]
    gid = lax.broadcasted_iota(jnp.int32, (ng, n_nodes), 0)
    oh = (gid == bvec).astype(jnp.float32)
    sums = oh @ h
    cnt = jnp.sum(oh, axis=1, keepdims=True)
    pooled = sums / jnp.maximum(cnt, 1.0)
    z = jnp.maximum(pooled @ pw1_ref[...] + pb1_ref[...], 0.0)
    o_ref[...] = z @ pw2_ref[...] + pb2_ref[...]


def _gather_call(h, src_r):
    ch = src_r.shape[0] * SPAN
    cpw = ch // NW
    nspan = cpw // SPAN
    mesh = plsc.VectorSubcoreMesh(core_axis_name="c", subcore_axis_name="s")

    @functools.partial(
        pl.kernel, mesh=mesh,
        out_type=jax.ShapeDtypeStruct((ch * CHUNK, 128), jnp.float32),
        compiler_params=pltpu.CompilerParams(use_tc_tiling_on_sc=False),
        scratch_types=[
            pltpu.VMEM((SPAN, CHUNK), jnp.int32),
            pltpu.VMEM((SPAN * CHUNK, H), jnp.float32),
            pltpu.SemaphoreType.DMA,
        ],
    )
    def k(h_hbm, src_hbm, out_hbm, idx_v, rows_v, sem):
        wid = lax.axis_index("s") * NC + lax.axis_index("c")
        for span in range(nspan):
            crow = wid * cpw + span * SPAN
            pltpu.sync_copy(src_hbm.at[wid * nspan + span], idx_v)
            cps = [pltpu.async_copy(h_hbm.at[idx_v.at[j]],
                                    rows_v.at[pl.ds(j * CHUNK, CHUNK)], sem)
                   for j in range(SPAN)]
            for cp in cps:
                cp.wait()
            pltpu.sync_copy(
                rows_v,
                out_hbm.at[pl.ds(crow * CHUNK, SPAN * CHUNK), pl.ds(0, H)])

    return k(h, src_r)


def _scatter_call(msg, dst_r, zeros_pad, npad):
    ch = dst_r.shape[0] * SPAN
    cpw = ch // NW
    nspan = cpw // SPAN
    zr = npad // NS
    mesh = plsc.VectorSubcoreMesh(core_axis_name="c", subcore_axis_name="s")

    @functools.partial(
        pl.kernel, mesh=mesh,
        out_type=jax.ShapeDtypeStruct((2 * npad, H), jnp.float32),
        compiler_params=pltpu.CompilerParams(use_tc_tiling_on_sc=False),
        scratch_types=[
            pltpu.VMEM((SPAN, CHUNK), jnp.int32),
            pltpu.VMEM((SPAN * CHUNK, H), jnp.float32),
            pltpu.VMEM_SHARED((npad, H), jnp.float32),
            pltpu.SemaphoreType.DMA,
        ],
    )
    def k(msg_hbm, dst_hbm, zero_hbm, out_hbm, idx_v, rows_v, acc, sem):
        cid = lax.axis_index("c")
        sid = lax.axis_index("s")
        wid = sid * NC + cid
        pltpu.sync_copy(zero_hbm.at[pl.ds(sid * zr, zr)],
                        acc.at[pl.ds(sid * zr, zr)])
        plsc.subcore_barrier()
        for span in range(nspan):
            crow = wid * cpw + span * SPAN
            pltpu.sync_copy(dst_hbm.at[wid * nspan + span], idx_v)
            pltpu.sync_copy(
                msg_hbm.at[pl.ds(crow * CHUNK, SPAN * CHUNK), pl.ds(0, H)],
                rows_v)
            cps = [pltpu.async_copy(rows_v.at[pl.ds(j * CHUNK, CHUNK)],
                                    acc.at[idx_v.at[j]], sem, add=True)
                   for j in range(SPAN)]
            for cp in cps:
                cp.wait()
        plsc.subcore_barrier()
        pltpu.sync_copy(acc.at[pl.ds(sid * zr, zr)],
                        out_hbm.at[pl.ds(cid * npad + sid * zr, zr)])

    return k(msg, dst_r, zeros_pad)


def kernel(x, edge_index, edge_attr, batch, emb_W, emb_b,
           c0_eW1, c0_eb1, c0_eW2, c0_eb2, c0_nW, c0_nb, bn0_g, bn0_b,
           c1_eW1, c1_eb1, c1_eW2, c1_eb2, c1_nW, c1_nb, bn1_g, bn1_b,
           p_W1, p_b1, p_W2, p_b2):
    n_nodes, node_dim = x.shape
    e_edges = edge_index.shape[1]
    edge_dim = edge_attr.shape[1]
    ng = 64
    out_dim = p_W2.shape[1]

    step = NW * SPAN
    ch = ((e_edges + CHUNK - 1) // CHUNK + step - 1) // step * step
    ep = ch * CHUNK
    npad = ((n_nodes + 1) + 255) // 256 * 256

    src = edge_index[0]
    dst = edge_index[1]
    pad_e = ep - e_edges
    src_r = jnp.concatenate(
        [src, jnp.zeros((pad_e,), jnp.int32)]).reshape(ch // SPAN, SPAN, CHUNK)
    dst_r = jnp.concatenate(
        [dst, jnp.full((pad_e,), n_nodes, jnp.int32)]).reshape(
            ch // SPAN, SPAN, CHUNK)
    ea_t = jnp.concatenate(
        [edge_attr, jnp.zeros((pad_e, edge_dim), jnp.float32)], axis=0).T
    zeros_pad = jnp.zeros((npad, H), jnp.float32)
    batch2 = batch.reshape(1, n_nodes)

    h = pl.pallas_call(
        _embed_body,
        out_shape=jax.ShapeDtypeStruct((n_nodes, H), jnp.float32),
    )(x, emb_W, emb_b.reshape(1, H))

    layers = [
        (c0_eW1, c0_eb1, c0_eW2, c0_eb2, c0_nW, c0_nb, bn0_g, bn0_b),
        (c1_eW1, c1_eb1, c1_eW2, c1_eb2, c1_nW, c1_nb, bn1_g, bn1_b),
    ]

    be = 2048
    grid_e = ep // be
    edge_call = pl.pallas_call(
        _edge_body,
        grid=(grid_e,),
        in_specs=[
            pl.BlockSpec((edge_dim, be), lambda i: (0, i)),
            pl.BlockSpec((be, 128), lambda i: (i, 0)),
            pl.BlockSpec((H, edge_dim), lambda i: (0, 0)),
            pl.BlockSpec((H, 1), lambda i: (0, 0)),
            pl.BlockSpec((H, H * H), lambda i: (0, 0)),
            pl.BlockSpec((H, H), lambda i: (0, 0)),
        ],
        out_specs=pl.BlockSpec((be, 128), lambda i: (i, 0)),
        out_shape=jax.ShapeDtypeStruct((ep, 128), jnp.float32),
    )

    for li, (ew1, eb1, ew2, eb2, nw, nb, g, b) in enumerate(layers):
        xj = _gather_call(h, src_r)
        w2ft = ew2.reshape(H, H, H).transpose(2, 0, 1).reshape(H, H * H)
        msg = edge_call(ea_t, xj, ew1.T, eb1.reshape(H, 1),
                        w2ft, eb2.reshape(H, H).T)
        acc = _scatter_call(msg, dst_r, zeros_pad, npad)
        if li == 0:
            h = pl.pallas_call(
                functools.partial(_node_body, n_nodes=n_nodes, npad=npad),
                out_shape=jax.ShapeDtypeStruct((n_nodes, H), jnp.float32),
            )(acc, nw, nb.reshape(1, H), g.reshape(1, H), b.reshape(1, H))
        else:
            out = pl.pallas_call(
                functools.partial(_final_body, n_nodes=n_nodes, npad=npad,
                                  ng=ng),
                out_shape=jax.ShapeDtypeStruct((ng, out_dim), jnp.float32),
            )(acc, nw, nb.reshape(1, H), g.reshape(1, H), b.reshape(1, H),
              batch2, p_W1, p_b1.reshape(1, -1), p_W2, p_b2.reshape(1, -1))
    return out


# speedup vs baseline: 6.2960x; 1.1098x over previous
import functools

import jax
import jax.numpy as jnp
from jax import lax
from jax.experimental import pallas as pl
from jax.experimental.pallas import tpu as pltpu
from jax.experimental.pallas import tpu_sc as plsc

H = 32
CHUNK = 128
SPAN = 20
NC = 2
NS = 16
NW = NC * NS


def _embed_body(x_ref, w_ref, b_ref, o_ref):
    o_ref[...] = x_ref[...] @ w_ref[...] + b_ref[...]


def _edge_body(eaT_ref, xj_ref, w1T_ref, b1T_ref, w2fT_ref, b2rT_ref, o_ref):
    xjT = xj_ref[:, :H].T
    ehT = jnp.maximum(w1T_ref[...] @ eaT_ref[...] + b1T_ref[...], 0.0)
    opT = jnp.concatenate([ehT[c:c + 1, :] * xjT for c in range(H)], axis=0)
    msgT = w2fT_ref[...] @ opT + b2rT_ref[...] @ xjT
    o_ref[:, :H] = msgT.T


def _node_common(acc_ref, nw_ref, nb_ref, g_ref, b_ref, n_nodes, npad):
    af = acc_ref[...]
    a = af[:n_nodes] + af[npad:npad + n_nodes]
    h = jnp.maximum(a @ nw_ref[...] + nb_ref[...], 0.0)
    mean = jnp.mean(h, axis=0, keepdims=True)
    var = jnp.mean((h - mean) ** 2, axis=0, keepdims=True)
    hn = (h - mean) / jnp.sqrt(var + 1e-5) * g_ref[...] + b_ref[...]
    return jnp.maximum(hn, 0.0)


def _node_body(acc_ref, nw_ref, nb_ref, g_ref, b_ref, o_ref, *, n_nodes, npad):
    o_ref[...] = _node_common(acc_ref, nw_ref, nb_ref, g_ref, b_ref,
                              n_nodes, npad)


def _final_body(acc_ref, nw_ref, nb_ref, g_ref, b_ref, batch_ref,
                pw1_ref, pb1_ref, pw2_ref, pb2_ref, o_ref, *,
                n_nodes, npad, ng):
    h = _node_common(acc_ref, nw_ref, nb_ref, g_ref, b_ref, n_nodes, npad)
    bvec = batch_ref[...]
    gid = lax.broadcasted_iota(jnp.int32, (ng, n_nodes), 0)
    oh = (gid == bvec).astype(jnp.float32)
    sums = oh @ h
    cnt = jnp.sum(oh, axis=1, keepdims=True)
    pooled = sums / jnp.maximum(cnt, 1.0)
    z = jnp.maximum(pooled @ pw1_ref[...] + pb1_ref[...], 0.0)
    o_ref[...] = z @ pw2_ref[...] + pb2_ref[...]


def _gather_call(h, src_r):
    ch = src_r.shape[0] * SPAN
    cpw = ch // NW
    nspan = cpw // SPAN
    mesh = plsc.VectorSubcoreMesh(core_axis_name="c", subcore_axis_name="s")

    @functools.partial(
        pl.kernel, mesh=mesh,
        out_type=jax.ShapeDtypeStruct((ch * CHUNK, 128), jnp.float32),
        compiler_params=pltpu.CompilerParams(use_tc_tiling_on_sc=False),
        scratch_types=[
            pltpu.VMEM((SPAN, CHUNK), jnp.int32),
            pltpu.VMEM((SPAN * CHUNK, H), jnp.float32),
            pltpu.SemaphoreType.DMA,
        ],
    )
    def k(h_hbm, src_hbm, out_hbm, idx_v, rows_v, sem):
        wid = lax.axis_index("s") * NC + lax.axis_index("c")
        for span in range(nspan):
            crow = wid * cpw + span * SPAN
            pltpu.sync_copy(src_hbm.at[wid * nspan + span], idx_v)
            cps = [pltpu.async_copy(h_hbm.at[idx_v.at[j]],
                                    rows_v.at[pl.ds(j * CHUNK, CHUNK)], sem)
                   for j in range(SPAN)]
            for cp in cps:
                cp.wait()
            pltpu.sync_copy(
                rows_v,
                out_hbm.at[pl.ds(crow * CHUNK, SPAN * CHUNK), pl.ds(0, H)])

    return k(h, src_r)


def _scatter_call(msg, dst_r, zeros_pad, npad):
    ch = dst_r.shape[0] * SPAN
    cpw = ch // NW
    nspan = cpw // SPAN
    zr = npad // NS
    mesh = plsc.VectorSubcoreMesh(core_axis_name="c", subcore_axis_name="s")

    @functools.partial(
        pl.kernel, mesh=mesh,
        out_type=jax.ShapeDtypeStruct((2 * npad, H), jnp.float32),
        compiler_params=pltpu.CompilerParams(use_tc_tiling_on_sc=False),
        scratch_types=[
            pltpu.VMEM((SPAN, CHUNK), jnp.int32),
            pltpu.VMEM((SPAN * CHUNK, H), jnp.float32),
            pltpu.VMEM_SHARED((npad, H), jnp.float32),
            pltpu.SemaphoreType.DMA,
        ],
    )
    def k(msg_hbm, dst_hbm, zero_hbm, out_hbm, idx_v, rows_v, acc, sem):
        cid = lax.axis_index("c")
        sid = lax.axis_index("s")
        wid = sid * NC + cid
        pltpu.sync_copy(zero_hbm.at[pl.ds(sid * zr, zr)],
                        acc.at[pl.ds(sid * zr, zr)])
        plsc.subcore_barrier()
        for span in range(nspan):
            crow = wid * cpw + span * SPAN
            pltpu.sync_copy(dst_hbm.at[wid * nspan + span], idx_v)
            pltpu.sync_copy(
                msg_hbm.at[pl.ds(crow * CHUNK, SPAN * CHUNK), pl.ds(0, H)],
                rows_v)
            cps = [pltpu.async_copy(rows_v.at[pl.ds(j * CHUNK, CHUNK)],
                                    acc.at[idx_v.at[j]], sem, add=True)
                   for j in range(SPAN)]
            for cp in cps:
                cp.wait()
        plsc.subcore_barrier()
        pltpu.sync_copy(acc.at[pl.ds(sid * zr, zr)],
                        out_hbm.at[pl.ds(cid * npad + sid * zr, zr)])

    return k(msg, dst_r, zeros_pad)


def kernel(x, edge_index, edge_attr, batch, emb_W, emb_b,
           c0_eW1, c0_eb1, c0_eW2, c0_eb2, c0_nW, c0_nb, bn0_g, bn0_b,
           c1_eW1, c1_eb1, c1_eW2, c1_eb2, c1_nW, c1_nb, bn1_g, bn1_b,
           p_W1, p_b1, p_W2, p_b2):
    n_nodes, node_dim = x.shape
    e_edges = edge_index.shape[1]
    edge_dim = edge_attr.shape[1]
    ng = 64
    out_dim = p_W2.shape[1]

    step = NW * SPAN
    ch = ((e_edges + CHUNK - 1) // CHUNK + step - 1) // step * step
    ep = ch * CHUNK
    npad = ((n_nodes + 1) + 255) // 256 * 256

    src = edge_index[0]
    dst = edge_index[1]
    pad_e = ep - e_edges
    src_r = jnp.concatenate(
        [src, jnp.zeros((pad_e,), jnp.int32)]).reshape(ch // SPAN, SPAN, CHUNK)
    dst_r = jnp.concatenate(
        [dst, jnp.full((pad_e,), n_nodes, jnp.int32)]).reshape(
            ch // SPAN, SPAN, CHUNK)
    ea_t = jnp.concatenate(
        [edge_attr, jnp.zeros((pad_e, edge_dim), jnp.float32)], axis=0).T
    zeros_pad = jnp.zeros((npad, H), jnp.float32)
    batch2 = batch.reshape(1, n_nodes)

    h = pl.pallas_call(
        _embed_body,
        out_shape=jax.ShapeDtypeStruct((n_nodes, H), jnp.float32),
    )(x, emb_W, emb_b.reshape(1, H))

    layers = [
        (c0_eW1, c0_eb1, c0_eW2, c0_eb2, c0_nW, c0_nb, bn0_g, bn0_b),
        (c1_eW1, c1_eb1, c1_eW2, c1_eb2, c1_nW, c1_nb, bn1_g, bn1_b),
    ]

    be = 4096
    grid_e = ep // be
    edge_call = pl.pallas_call(
        _edge_body,
        grid=(grid_e,),
        in_specs=[
            pl.BlockSpec((edge_dim, be), lambda i: (0, i)),
            pl.BlockSpec((be, 128), lambda i: (i, 0)),
            pl.BlockSpec((H, edge_dim), lambda i: (0, 0)),
            pl.BlockSpec((H, 1), lambda i: (0, 0)),
            pl.BlockSpec((H, H * H), lambda i: (0, 0)),
            pl.BlockSpec((H, H), lambda i: (0, 0)),
        ],
        out_specs=pl.BlockSpec((be, 128), lambda i: (i, 0)),
        out_shape=jax.ShapeDtypeStruct((ep, 128), jnp.float32),
    )

    for li, (ew1, eb1, ew2, eb2, nw, nb, g, b) in enumerate(layers):
        xj = _gather_call(h, src_r)
        w2ft = ew2.reshape(H, H, H).transpose(2, 0, 1).reshape(H, H * H)
        msg = edge_call(ea_t, xj, ew1.T, eb1.reshape(H, 1),
                        w2ft, eb2.reshape(H, H).T)
        acc = _scatter_call(msg, dst_r, zeros_pad, npad)
        if li == 0:
            h = pl.pallas_call(
                functools.partial(_node_body, n_nodes=n_nodes, npad=npad),
                out_shape=jax.ShapeDtypeStruct((n_nodes, H), jnp.float32),
            )(acc, nw, nb.reshape(1, H), g.reshape(1, H), b.reshape(1, H))
        else:
            out = pl.pallas_call(
                functools.partial(_final_body, n_nodes=n_nodes, npad=npad,
                                  ng=ng),
                out_shape=jax.ShapeDtypeStruct((ng, out_dim), jnp.float32),
            )(acc, nw, nb.reshape(1, H), g.reshape(1, H), b.reshape(1, H),
              batch2, p_W1, p_b1.reshape(1, -1), p_W2, p_b2.reshape(1, -1))
    return out


# speedup vs baseline: 6.5095x; 1.0339x over previous
import functools

import jax
import jax.numpy as jnp
from jax import lax
from jax.experimental import pallas as pl
from jax.experimental.pallas import tpu as pltpu
from jax.experimental.pallas import tpu_sc as plsc

H = 32
CHUNK = 128
SPAN = 20
NC = 2
NS = 16
NW = NC * NS


def _embed_body(x_ref, w_ref, b_ref, o_ref):
    o_ref[...] = x_ref[...] @ w_ref[...] + b_ref[...]


def _edge_body(eaT_ref, xj_ref, w1T_ref, b1T_ref, w2fT_ref, b2rT_ref, o_ref):
    xjT = xj_ref[:, :H].T
    ehT = jnp.maximum(w1T_ref[...] @ eaT_ref[...] + b1T_ref[...], 0.0)
    opT = jnp.concatenate([ehT[c:c + 1, :] * xjT for c in range(H)], axis=0)
    msgT = w2fT_ref[...] @ opT + b2rT_ref[...] @ xjT
    o_ref[:, :H] = msgT.T


def _node_common(acca_ref, accb_ref, nw_ref, nb_ref, g_ref, b_ref,
                 n_nodes, npad):
    aa = acca_ref[...]
    ab = accb_ref[...]
    a = (aa[:n_nodes] + aa[npad:npad + n_nodes]
         + ab[:n_nodes] + ab[npad:npad + n_nodes])
    h = jnp.maximum(a @ nw_ref[...] + nb_ref[...], 0.0)
    mean = jnp.mean(h, axis=0, keepdims=True)
    var = jnp.mean((h - mean) ** 2, axis=0, keepdims=True)
    hn = (h - mean) / jnp.sqrt(var + 1e-5) * g_ref[...] + b_ref[...]
    return jnp.maximum(hn, 0.0)


def _node_body(acca_ref, accb_ref, nw_ref, nb_ref, g_ref, b_ref, o_ref, *,
               n_nodes, npad):
    o_ref[...] = _node_common(acca_ref, accb_ref, nw_ref, nb_ref, g_ref,
                              b_ref, n_nodes, npad)


def _final_body(acca_ref, accb_ref, nw_ref, nb_ref, g_ref, b_ref, batch_ref,
                pw1_ref, pb1_ref, pw2_ref, pb2_ref, o_ref, *,
                n_nodes, npad, ng):
    h = _node_common(acca_ref, accb_ref, nw_ref, nb_ref, g_ref, b_ref,
                     n_nodes, npad)
    bvec = batch_ref[...]
    gid = lax.broadcasted_iota(jnp.int32, (ng, n_nodes), 0)
    oh = (gid == bvec).astype(jnp.float32)
    sums = oh @ h
    cnt = jnp.sum(oh, axis=1, keepdims=True)
    pooled = sums / jnp.maximum(cnt, 1.0)
    z = jnp.maximum(pooled @ pw1_ref[...] + pb1_ref[...], 0.0)
    o_ref[...] = z @ pw2_ref[...] + pb2_ref[...]


def _gather_call(h, src_r):
    ch = src_r.shape[0] * SPAN
    cpw = ch // NW
    nspan = cpw // SPAN
    mesh = plsc.VectorSubcoreMesh(core_axis_name="c", subcore_axis_name="s")

    @functools.partial(
        pl.kernel, mesh=mesh,
        out_type=jax.ShapeDtypeStruct((ch * CHUNK, 128), jnp.float32),
        compiler_params=pltpu.CompilerParams(use_tc_tiling_on_sc=False),
        scratch_types=[
            pltpu.VMEM((SPAN, CHUNK), jnp.int32),
            pltpu.VMEM((SPAN * CHUNK, H), jnp.float32),
            pltpu.SemaphoreType.DMA,
        ],
    )
    def k(h_hbm, src_hbm, out_hbm, idx_v, rows_v, sem):
        wid = lax.axis_index("s") * NC + lax.axis_index("c")
        for span in range(nspan):
            crow = wid * cpw + span * SPAN
            pltpu.sync_copy(src_hbm.at[wid * nspan + span], idx_v)
            cps = [pltpu.async_copy(h_hbm.at[idx_v.at[j]],
                                    rows_v.at[pl.ds(j * CHUNK, CHUNK)], sem)
                   for j in range(SPAN)]
            for cp in cps:
                cp.wait()
            pltpu.sync_copy(
                rows_v,
                out_hbm.at[pl.ds(crow * CHUNK, SPAN * CHUNK), pl.ds(0, H)])

    return k(h, src_r)


def _scatter_call(msg, dst_r, zeros_pad, npad):
    ch = dst_r.shape[0] * SPAN
    cpw = ch // NW
    nspan = cpw // SPAN
    zr = npad // NS
    mesh = plsc.VectorSubcoreMesh(core_axis_name="c", subcore_axis_name="s")

    @functools.partial(
        pl.kernel, mesh=mesh,
        out_type=jax.ShapeDtypeStruct((2 * npad, H), jnp.float32),
        compiler_params=pltpu.CompilerParams(use_tc_tiling_on_sc=False),
        scratch_types=[
            pltpu.VMEM((SPAN, CHUNK), jnp.int32),
            pltpu.VMEM((SPAN * CHUNK, H), jnp.float32),
            pltpu.VMEM_SHARED((npad, H), jnp.float32),
            pltpu.SemaphoreType.DMA,
        ],
    )
    def k(msg_hbm, dst_hbm, zero_hbm, out_hbm, idx_v, rows_v, acc, sem):
        cid = lax.axis_index("c")
        sid = lax.axis_index("s")
        wid = sid * NC + cid
        pltpu.sync_copy(zero_hbm.at[pl.ds(sid * zr, zr)],
                        acc.at[pl.ds(sid * zr, zr)])
        plsc.subcore_barrier()
        for span in range(nspan):
            crow = wid * cpw + span * SPAN
            pltpu.sync_copy(dst_hbm.at[wid * nspan + span], idx_v)
            pltpu.sync_copy(
                msg_hbm.at[pl.ds(crow * CHUNK, SPAN * CHUNK), pl.ds(0, H)],
                rows_v)
            cps = [pltpu.async_copy(rows_v.at[pl.ds(j * CHUNK, CHUNK)],
                                    acc.at[idx_v.at[j]], sem, add=True)
                   for j in range(SPAN)]
            for cp in cps:
                cp.wait()
        plsc.subcore_barrier()
        pltpu.sync_copy(acc.at[pl.ds(sid * zr, zr)],
                        out_hbm.at[pl.ds(cid * npad + sid * zr, zr)])

    return k(msg, dst_r, zeros_pad)


def kernel(x, edge_index, edge_attr, batch, emb_W, emb_b,
           c0_eW1, c0_eb1, c0_eW2, c0_eb2, c0_nW, c0_nb, bn0_g, bn0_b,
           c1_eW1, c1_eb1, c1_eW2, c1_eb2, c1_nW, c1_nb, bn1_g, bn1_b,
           p_W1, p_b1, p_W2, p_b2):
    n_nodes, node_dim = x.shape
    e_edges = edge_index.shape[1]
    edge_dim = edge_attr.shape[1]
    ng = 64
    out_dim = p_W2.shape[1]

    step = 2 * NW * SPAN
    ch = ((e_edges + CHUNK - 1) // CHUNK + step - 1) // step * step
    ep = ch * CHUNK
    npad = ((n_nodes + 1) + 255) // 256 * 256

    src = edge_index[0]
    dst = edge_index[1]
    pad_e = ep - e_edges
    src_r = jnp.concatenate(
        [src, jnp.zeros((pad_e,), jnp.int32)]).reshape(ch // SPAN, SPAN, CHUNK)
    dst_r = jnp.concatenate(
        [dst, jnp.full((pad_e,), n_nodes, jnp.int32)]).reshape(
            ch // SPAN, SPAN, CHUNK)
    ea_t = jnp.concatenate(
        [edge_attr, jnp.zeros((pad_e, edge_dim), jnp.float32)], axis=0).T
    zeros_pad = jnp.zeros((npad, H), jnp.float32)
    batch2 = batch.reshape(1, n_nodes)

    h = pl.pallas_call(
        _embed_body,
        out_shape=jax.ShapeDtypeStruct((n_nodes, H), jnp.float32),
    )(x, emb_W, emb_b.reshape(1, H))

    layers = [
        (c0_eW1, c0_eb1, c0_eW2, c0_eb2, c0_nW, c0_nb, bn0_g, bn0_b),
        (c1_eW1, c1_eb1, c1_eW2, c1_eb2, c1_nW, c1_nb, bn1_g, bn1_b),
    ]

    be = 4096
    eph = ep // 2
    grid_e = eph // be
    edge_call = pl.pallas_call(
        _edge_body,
        grid=(grid_e,),
        in_specs=[
            pl.BlockSpec((edge_dim, be), lambda i: (0, i)),
            pl.BlockSpec((be, 128), lambda i: (i, 0)),
            pl.BlockSpec((H, edge_dim), lambda i: (0, 0)),
            pl.BlockSpec((H, 1), lambda i: (0, 0)),
            pl.BlockSpec((H, H * H), lambda i: (0, 0)),
            pl.BlockSpec((H, H), lambda i: (0, 0)),
        ],
        out_specs=pl.BlockSpec((be, 128), lambda i: (i, 0)),
        out_shape=jax.ShapeDtypeStruct((eph, 128), jnp.float32),
    )

    hg = ch // SPAN // 2
    src_halves = (src_r[:hg], src_r[hg:])
    dst_halves = (dst_r[:hg], dst_r[hg:])
    ea_halves = (ea_t[:, :eph], ea_t[:, eph:])

    for li, (ew1, eb1, ew2, eb2, nw, nb, g, b) in enumerate(layers):
        w2ft = ew2.reshape(H, H, H).transpose(2, 0, 1).reshape(H, H * H)
        accs = []
        for hf in range(2):
            xj = _gather_call(h, src_halves[hf])
            msg = edge_call(ea_halves[hf], xj, ew1.T, eb1.reshape(H, 1),
                            w2ft, eb2.reshape(H, H).T)
            accs.append(_scatter_call(msg, dst_halves[hf], zeros_pad, npad))
        if li == 0:
            h = pl.pallas_call(
                functools.partial(_node_body, n_nodes=n_nodes, npad=npad),
                out_shape=jax.ShapeDtypeStruct((n_nodes, H), jnp.float32),
            )(accs[0], accs[1], nw, nb.reshape(1, H), g.reshape(1, H),
              b.reshape(1, H))
        else:
            out = pl.pallas_call(
                functools.partial(_final_body, n_nodes=n_nodes, npad=npad,
                                  ng=ng),
                out_shape=jax.ShapeDtypeStruct((ng, out_dim), jnp.float32),
            )(accs[0], accs[1], nw, nb.reshape(1, H), g.reshape(1, H),
              b.reshape(1, H), batch2, p_W1, p_b1.reshape(1, -1), p_W2,
              p_b2.reshape(1, -1))
    return out


# speedup vs baseline: 6.8876x; 1.0581x over previous
import functools

import jax
import jax.numpy as jnp
from jax import lax
from jax.experimental import pallas as pl
from jax.experimental.pallas import tpu as pltpu
from jax.experimental.pallas import tpu_sc as plsc

H = 32
CHUNK = 128
SPAN = 20
NC = 2
NS = 16
NW = NC * NS


def _embed_body(x_ref, w_ref, b_ref, o_ref):
    o_ref[...] = x_ref[...] @ w_ref[...] + b_ref[...]


def _edge_body(eaT_ref, xj_ref, w1T_ref, b1T_ref, w2fT_ref, b2rT_ref, o_ref):
    xjT = xj_ref[:, :H].T
    ehT = jnp.maximum(w1T_ref[...] @ eaT_ref[...] + b1T_ref[...], 0.0)
    opT = jnp.concatenate([ehT[c:c + 1, :] * xjT for c in range(H)], axis=0)
    msgT = w2fT_ref[...] @ opT + b2rT_ref[...] @ xjT
    o_ref[:, :H] = msgT.T


def _node_common(acca_ref, accb_ref, nw_ref, nb_ref, g_ref, b_ref,
                 n_nodes, npad):
    aa = acca_ref[:, :H]
    ab = accb_ref[:, :H]
    a = (aa[:n_nodes] + aa[npad:npad + n_nodes]
         + ab[:n_nodes] + ab[npad:npad + n_nodes])
    h = jnp.maximum(a @ nw_ref[...] + nb_ref[...], 0.0)
    mean = jnp.mean(h, axis=0, keepdims=True)
    var = jnp.mean((h - mean) ** 2, axis=0, keepdims=True)
    hn = (h - mean) / jnp.sqrt(var + 1e-5) * g_ref[...] + b_ref[...]
    return jnp.maximum(hn, 0.0)


def _node_body(acca_ref, accb_ref, nw_ref, nb_ref, g_ref, b_ref, o_ref, *,
               n_nodes, npad):
    o_ref[...] = _node_common(acca_ref, accb_ref, nw_ref, nb_ref, g_ref,
                              b_ref, n_nodes, npad)


def _final_body(acca_ref, accb_ref, nw_ref, nb_ref, g_ref, b_ref, batch_ref,
                pw1_ref, pb1_ref, pw2_ref, pb2_ref, o_ref, *,
                n_nodes, npad, ng):
    h = _node_common(acca_ref, accb_ref, nw_ref, nb_ref, g_ref, b_ref,
                     n_nodes, npad)
    bvec = batch_ref[...]
    gid = lax.broadcasted_iota(jnp.int32, (ng, n_nodes), 0)
    oh = (gid == bvec).astype(jnp.float32)
    sums = oh @ h
    cnt = jnp.sum(oh, axis=1, keepdims=True)
    pooled = sums / jnp.maximum(cnt, 1.0)
    z = jnp.maximum(pooled @ pw1_ref[...] + pb1_ref[...], 0.0)
    o_ref[...] = z @ pw2_ref[...] + pb2_ref[...]


def _gather_call(h, src_r):
    ch = src_r.shape[0] * SPAN
    cpw = ch // NW
    nspan = cpw // SPAN
    mesh = plsc.VectorSubcoreMesh(core_axis_name="c", subcore_axis_name="s")

    @functools.partial(
        pl.kernel, mesh=mesh,
        out_type=jax.ShapeDtypeStruct((ch * CHUNK, 128), jnp.float32),
        compiler_params=pltpu.CompilerParams(use_tc_tiling_on_sc=False),
        scratch_types=[
            pltpu.VMEM((SPAN, CHUNK), jnp.int32),
            pltpu.VMEM((SPAN * CHUNK, H), jnp.float32),
            pltpu.SemaphoreType.DMA,
        ],
    )
    def k(h_hbm, src_hbm, out_hbm, idx_v, rows_v, sem):
        wid = lax.axis_index("s") * NC + lax.axis_index("c")
        for span in range(nspan):
            crow = wid * cpw + span * SPAN
            pltpu.sync_copy(src_hbm.at[wid * nspan + span], idx_v)
            cps = [pltpu.async_copy(h_hbm.at[idx_v.at[j]],
                                    rows_v.at[pl.ds(j * CHUNK, CHUNK)], sem)
                   for j in range(SPAN)]
            for cp in cps:
                cp.wait()
            pltpu.sync_copy(
                rows_v,
                out_hbm.at[pl.ds(crow * CHUNK, SPAN * CHUNK), pl.ds(0, H)])

    return k(h, src_r)


def _scatter_call(msg, dst_r, zeros_pad, npad):
    ch = dst_r.shape[0] * SPAN
    cpw = ch // NW
    nspan = cpw // SPAN
    zr = npad // NS
    mesh = plsc.VectorSubcoreMesh(core_axis_name="c", subcore_axis_name="s")

    @functools.partial(
        pl.kernel, mesh=mesh,
        out_type=jax.ShapeDtypeStruct((2 * npad, 128), jnp.float32),
        compiler_params=pltpu.CompilerParams(use_tc_tiling_on_sc=False),
        scratch_types=[
            pltpu.VMEM((SPAN, CHUNK), jnp.int32),
            pltpu.VMEM((SPAN * CHUNK, H), jnp.float32),
            pltpu.VMEM_SHARED((npad, H), jnp.float32),
            pltpu.SemaphoreType.DMA,
        ],
    )
    def k(msg_hbm, dst_hbm, zero_hbm, out_hbm, idx_v, rows_v, acc, sem):
        cid = lax.axis_index("c")
        sid = lax.axis_index("s")
        wid = sid * NC + cid
        pltpu.sync_copy(zero_hbm.at[pl.ds(sid * zr, zr)],
                        acc.at[pl.ds(sid * zr, zr)])
        plsc.subcore_barrier()
        for span in range(nspan):
            crow = wid * cpw + span * SPAN
            pltpu.sync_copy(dst_hbm.at[wid * nspan + span], idx_v)
            pltpu.sync_copy(
                msg_hbm.at[pl.ds(crow * CHUNK, SPAN * CHUNK), pl.ds(0, H)],
                rows_v)
            cps = [pltpu.async_copy(rows_v.at[pl.ds(j * CHUNK, CHUNK)],
                                    acc.at[idx_v.at[j]], sem, add=True)
                   for j in range(SPAN)]
            for cp in cps:
                cp.wait()
        plsc.subcore_barrier()
        pltpu.sync_copy(acc.at[pl.ds(sid * zr, zr)],
                        out_hbm.at[pl.ds(cid * npad + sid * zr, zr),
                                   pl.ds(0, H)])

    return k(msg, dst_r, zeros_pad)


def kernel(x, edge_index, edge_attr, batch, emb_W, emb_b,
           c0_eW1, c0_eb1, c0_eW2, c0_eb2, c0_nW, c0_nb, bn0_g, bn0_b,
           c1_eW1, c1_eb1, c1_eW2, c1_eb2, c1_nW, c1_nb, bn1_g, bn1_b,
           p_W1, p_b1, p_W2, p_b2):
    n_nodes, node_dim = x.shape
    e_edges = edge_index.shape[1]
    edge_dim = edge_attr.shape[1]
    ng = 64
    out_dim = p_W2.shape[1]

    step = 2 * NW * SPAN
    ch = ((e_edges + CHUNK - 1) // CHUNK + step - 1) // step * step
    ep = ch * CHUNK
    npad = ((n_nodes + 1) + 255) // 256 * 256

    src = edge_index[0]
    dst = edge_index[1]
    pad_e = ep - e_edges
    src_r = jnp.concatenate(
        [src, jnp.zeros((pad_e,), jnp.int32)]).reshape(ch // SPAN, SPAN, CHUNK)
    dst_r = jnp.concatenate(
        [dst, jnp.full((pad_e,), n_nodes, jnp.int32)]).reshape(
            ch // SPAN, SPAN, CHUNK)
    ea_t = jnp.concatenate(
        [edge_attr, jnp.zeros((pad_e, edge_dim), jnp.float32)], axis=0).T
    zeros_pad = jnp.zeros((npad, H), jnp.float32)
    batch2 = batch.reshape(1, n_nodes)

    h = pl.pallas_call(
        _embed_body,
        out_shape=jax.ShapeDtypeStruct((n_nodes, H), jnp.float32),
    )(x, emb_W, emb_b.reshape(1, H))

    layers = [
        (c0_eW1, c0_eb1, c0_eW2, c0_eb2, c0_nW, c0_nb, bn0_g, bn0_b),
        (c1_eW1, c1_eb1, c1_eW2, c1_eb2, c1_nW, c1_nb, bn1_g, bn1_b),
    ]

    be = 4096
    eph = ep // 2
    grid_e = eph // be
    edge_call = pl.pallas_call(
        _edge_body,
        grid=(grid_e,),
        in_specs=[
            pl.BlockSpec((edge_dim, be), lambda i: (0, i)),
            pl.BlockSpec((be, 128), lambda i: (i, 0)),
            pl.BlockSpec((H, edge_dim), lambda i: (0, 0)),
            pl.BlockSpec((H, 1), lambda i: (0, 0)),
            pl.BlockSpec((H, H * H), lambda i: (0, 0)),
            pl.BlockSpec((H, H), lambda i: (0, 0)),
        ],
        out_specs=pl.BlockSpec((be, 128), lambda i: (i, 0)),
        out_shape=jax.ShapeDtypeStruct((eph, 128), jnp.float32),
    )

    hg = ch // SPAN // 2
    src_halves = (src_r[:hg], src_r[hg:])
    dst_halves = (dst_r[:hg], dst_r[hg:])
    ea_halves = (ea_t[:, :eph], ea_t[:, eph:])

    for li, (ew1, eb1, ew2, eb2, nw, nb, g, b) in enumerate(layers):
        w2ft = ew2.reshape(H, H, H).transpose(2, 0, 1).reshape(H, H * H)
        accs = []
        for hf in range(2):
            xj = _gather_call(h, src_halves[hf])
            msg = edge_call(ea_halves[hf], xj, ew1.T, eb1.reshape(H, 1),
                            w2ft, eb2.reshape(H, H).T)
            accs.append(_scatter_call(msg, dst_halves[hf], zeros_pad, npad))
        if li == 0:
            h = pl.pallas_call(
                functools.partial(_node_body, n_nodes=n_nodes, npad=npad),
                out_shape=jax.ShapeDtypeStruct((n_nodes, H), jnp.float32),
            )(accs[0], accs[1], nw, nb.reshape(1, H), g.reshape(1, H),
              b.reshape(1, H))
        else:
            out = pl.pallas_call(
                functools.partial(_final_body, n_nodes=n_nodes, npad=npad,
                                  ng=ng),
                out_shape=jax.ShapeDtypeStruct((ng, out_dim), jnp.float32),
            )(accs[0], accs[1], nw, nb.reshape(1, H), g.reshape(1, H),
              b.reshape(1, H), batch2, p_W1, p_b1.reshape(1, -1), p_W2,
              p_b2.reshape(1, -1))
    return out
